# baseline (speedup 1.0000x reference)
import jax
import jax.numpy as jnp
from jax import lax
from jax.experimental import pallas as pl
from jax.experimental.pallas import tpu as pltpu
from jax.experimental.pallas import tpu_sc as plsc

DIM = 32
L = 16
NC = 2
NS = 16
NW = NC * NS

NUM_EMB = 1000000
N_TC = NUM_EMB // 128
TC_PER_W = N_TC // NW + 1
PACKED_ROWS = NUM_EMB // 8
TAIL_BASE = N_TC * 128

N_ROWS = 4096
N_COLS = 200
ROWS_PER_W = N_ROWS // NW
R = 8
N_CHUNKS = ROWS_PER_W // R
CR = R * N_COLS


def _cast_body(wt_hbm, tail_hbm, out_hbm, t_v, o_v, isem, osem):
    wid = lax.axis_index("s") * NC + lax.axis_index("c")

    evens = lax.iota(jnp.int32, L) * 2
    odds = evens + 1

    def tcg(t):
        return t * NW + wid

    def fire(t):
        p = t % 2

        @pl.when(tcg(t) < N_TC)
        def _():
            col0 = pl.multiple_of(tcg(t) * 128, 128)
            pltpu.async_copy(
                wt_hbm.at[:, pl.ds(col0, 128)], t_v.at[p], isem.at[p]
            )

    def wait_in(t):
        p = t % 2

        @pl.when(tcg(t) < N_TC)
        def _():
            pltpu.make_async_copy(
                wt_hbm.at[:, pl.ds(0, 128)], t_v.at[p], isem.at[p]
            ).wait()

    def convert(t, n_lanes):
        p = t % 2

        def cvt(l, _):
            pv = jnp.full((L,), p, dtype=jnp.int32)
            lv = jnp.full((L,), l, dtype=jnp.int32)
            a = plsc.load_gather(t_v, [pv, evens, lv])
            b = plsc.load_gather(t_v, [pv, odds, lv])
            pk = plsc.pack(a, b, format=plsc.PackFormat.INTERLEAVED)
            o_v[p, l // 8, pl.ds((l % 8) * L, L)] = plsc.bitcast(pk, jnp.int32)
            return 0

        lax.fori_loop(0, n_lanes, cvt, 0, unroll=8)

    def out_copy(t):
        p = t % 2
        row0 = pl.multiple_of(tcg(t) * 16, 8)
        return pltpu.make_async_copy(
            o_v.at[p], out_hbm.at[pl.ds(row0, 16)], osem.at[p]
        )

    def step(t, _):
        fire(t + 1)
        wait_in(t)

        @pl.when(tcg(t) >= 2 * NW)
        def _():
            out_copy(t - 2).wait()

        @pl.when(tcg(t) < N_TC)
        def _():
            convert(t, 128)
            out_copy(t).start()

        return 0

    fire(0)
    lax.fori_loop(0, TC_PER_W, step, 0)

    @pl.when(tcg(TC_PER_W - 2) < N_TC)
    def _():
        out_copy(TC_PER_W - 2).wait()

    @pl.when(tcg(TC_PER_W - 1) < N_TC)
    def _():
        out_copy(TC_PER_W - 1).wait()

    @pl.when(wid == 4)
    def _():
        pltpu.sync_copy(tail_hbm, o_v.at[0, pl.ds(0, 8)])
        pltpu.sync_copy(
            o_v.at[0, pl.ds(0, 8)], out_hbm.at[pl.ds(PACKED_ROWS - 8, 8)]
        )


def _gather_body(idx_hbm, table_hbm, out_hbm, idx_v, out_v, gsem, osem):
    wid = lax.axis_index("s") * NC + lax.axis_index("c")
    base = wid * ROWS_PER_W

    def stage_and_fire(g):
        p = g % 2
        row0 = base + g * R
        pltpu.sync_copy(idx_hbm.at[pl.ds(row0, R)], idx_v.at[p])
        for rr in range(R):
            pltpu.async_copy(
                table_hbm.at[idx_v.at[p, rr, pl.ds(0, 128)]],
                out_v.at[p, rr, pl.ds(0, 128)],
                gsem.at[p],
            )
            pltpu.async_copy(
                table_hbm.at[idx_v.at[p, rr, pl.ds(128, 72)]],
                out_v.at[p, rr, pl.ds(128, 72)],
                gsem.at[p],
            )

    def drain_gathers(g):
        p = g % 2
        pltpu.make_async_copy(
            table_hbm.at[pl.ds(0, CR)],
            out_v.at[p],
            gsem.at[p],
        ).wait()

    def out_copy(g):
        p = g % 2
        row0 = base + g * R
        return pltpu.make_async_copy(
            out_v.at[p], out_hbm.at[pl.ds(row0, R)], osem.at[p]
        )

    def step(g, _):
        @pl.when(g + 1 < N_CHUNKS)
        def _():
            stage_and_fire(g + 1)

        drain_gathers(g)

        @pl.when(g >= 2)
        def _():
            out_copy(g - 2).wait()

        out_copy(g).start()
        return 0

    stage_and_fire(0)
    lax.fori_loop(0, N_CHUNKS, step, 0)
    out_copy(N_CHUNKS - 2).wait()
    out_copy(N_CHUNKS - 1).wait()


@jax.jit
def _impl(idx, wt):
    mesh = plsc.VectorSubcoreMesh(core_axis_name="c", subcore_axis_name="s")

    tail16 = jax.lax.bitcast_convert_type(
        wt[:, TAIL_BASE:].T.astype(jnp.bfloat16).reshape(64, 16, 2),
        jnp.int32,
    ).reshape(8, 128)

    packed = pl.kernel(
        _cast_body,
        out_type=jax.ShapeDtypeStruct((PACKED_ROWS, 128), jnp.int32),
        mesh=mesh,
        scratch_types=[
            pltpu.VMEM((2, DIM, 128), jnp.float32),
            pltpu.VMEM((2, 16, 128), jnp.int32),
            pltpu.SemaphoreType.DMA((2,)),
            pltpu.SemaphoreType.DMA((2,)),
        ],
        compiler_params=pltpu.CompilerParams(
            needs_layout_passes=False, use_tc_tiling_on_sc=True
        ),
    )(wt, tail16)

    rows16 = packed.reshape(NUM_EMB, 16)

    out_i32 = pl.kernel(
        _gather_body,
        out_type=jax.ShapeDtypeStruct((N_ROWS, N_COLS, 16), jnp.int32),
        mesh=mesh,
        scratch_types=[
            pltpu.VMEM((2, R, N_COLS), jnp.int32),
            pltpu.VMEM((2, R, N_COLS, 16), jnp.int32),
            pltpu.SemaphoreType.DMA((2,)),
            pltpu.SemaphoreType.DMA((2,)),
        ],
        compiler_params=pltpu.CompilerParams(
            needs_layout_passes=False, use_tc_tiling_on_sc=False
        ),
    )(idx, rows16)

    out_pairs = jax.lax.bitcast_convert_type(out_i32, jnp.bfloat16)
    return out_pairs.reshape(N_ROWS, N_COLS, DIM)


def kernel(input, embedding_weight):
    return _impl(input.astype(jnp.int32), embedding_weight.T)


# speedup vs baseline: 1.4169x; 1.4169x over previous
import jax
import jax.numpy as jnp
from jax import lax
from jax.experimental import pallas as pl
from jax.experimental.pallas import tpu as pltpu
from jax.experimental.pallas import tpu_sc as plsc

DIM = 32
L = 16
NC = 2
NS = 16
NW = NC * NS

NUM_EMB = 1000000
N_TC = NUM_EMB // 128
TC_PER_W = N_TC // NW + 1
PACKED_ROWS = NUM_EMB // 8
TAIL_BASE = N_TC * 128

N_ROWS = 4096
N_COLS = 200
ROWS_PER_W = N_ROWS // NW
R = 8
N_CHUNKS = ROWS_PER_W // R
CR = R * N_COLS


def _cast_body(wt_hbm, tail_hbm, out_hbm, t_v, o_v, isem, osem):
    wid = lax.axis_index("s") * NC + lax.axis_index("c")

    lanes = lax.iota(jnp.int32, L)
    row_half = lanes // 8
    col_base = (lanes % 8) * L
    colvs = [col_base + k for k in range(L)]

    def tcg(t):
        return t * NW + wid

    def fire(t):
        p = t % 2

        @pl.when(tcg(t) < N_TC)
        def _():
            col0 = pl.multiple_of(tcg(t) * 128, 128)
            pltpu.async_copy(
                wt_hbm.at[:, pl.ds(col0, 128)], t_v.at[p], isem.at[p]
            )

    def wait_in(t):
        p = t % 2

        @pl.when(tcg(t) < N_TC)
        def _():
            pltpu.make_async_copy(
                wt_hbm.at[:, pl.ds(0, 128)], t_v.at[p], isem.at[p]
            ).wait()

    def convert(t):
        p = t % 2
        ov = o_v.at[p]

        def lgbody(lg, _):
            c0 = pl.multiple_of(lg * L, L)
            rowv = row_half + 2 * lg
            for k in range(L):
                a = t_v[p, 2 * k, pl.ds(c0, L)]
                b = t_v[p, 2 * k + 1, pl.ds(c0, L)]
                pk = plsc.pack(a, b, format=plsc.PackFormat.INTERLEAVED)
                plsc.store_scatter(
                    ov, [rowv, colvs[k]], plsc.bitcast(pk, jnp.int32)
                )
            return 0

        lax.fori_loop(0, 8, lgbody, 0)

    def out_copy(t):
        p = t % 2
        row0 = pl.multiple_of(tcg(t) * 16, 8)
        return pltpu.make_async_copy(
            o_v.at[p], out_hbm.at[pl.ds(row0, 16)], osem.at[p]
        )

    def step(t, _):
        fire(t + 1)
        wait_in(t)

        @pl.when(tcg(t) >= 2 * NW)
        def _():
            out_copy(t - 2).wait()

        @pl.when(tcg(t) < N_TC)
        def _():
            convert(t)
            out_copy(t).start()

        return 0

    fire(0)
    lax.fori_loop(0, TC_PER_W, step, 0)

    @pl.when(tcg(TC_PER_W - 2) < N_TC)
    def _():
        out_copy(TC_PER_W - 2).wait()

    @pl.when(tcg(TC_PER_W - 1) < N_TC)
    def _():
        out_copy(TC_PER_W - 1).wait()

    @pl.when(wid == 4)
    def _():
        pltpu.sync_copy(tail_hbm, o_v.at[0, pl.ds(0, 8)])
        pltpu.sync_copy(
            o_v.at[0, pl.ds(0, 8)], out_hbm.at[pl.ds(PACKED_ROWS - 8, 8)]
        )


def _gather_body(idx_hbm, table_hbm, out_hbm, idx_v, out_v, gsem, osem):
    wid = lax.axis_index("s") * NC + lax.axis_index("c")
    base = wid * ROWS_PER_W

    def stage_and_fire(g):
        p = g % 2
        row0 = base + g * R
        pltpu.sync_copy(idx_hbm.at[pl.ds(row0, R)], idx_v.at[p])
        for rr in range(R):
            pltpu.async_copy(
                table_hbm.at[idx_v.at[p, rr, pl.ds(0, 128)]],
                out_v.at[p, rr, pl.ds(0, 128)],
                gsem.at[p],
            )
            pltpu.async_copy(
                table_hbm.at[idx_v.at[p, rr, pl.ds(128, 72)]],
                out_v.at[p, rr, pl.ds(128, 72)],
                gsem.at[p],
            )

    def drain_gathers(g):
        p = g % 2
        pltpu.make_async_copy(
            table_hbm.at[pl.ds(0, CR)],
            out_v.at[p],
            gsem.at[p],
        ).wait()

    def out_copy(g):
        p = g % 2
        row0 = base + g * R
        return pltpu.make_async_copy(
            out_v.at[p], out_hbm.at[pl.ds(row0, R)], osem.at[p]
        )

    def step(g, _):
        @pl.when(g + 1 < N_CHUNKS)
        def _():
            stage_and_fire(g + 1)

        drain_gathers(g)

        @pl.when(g >= 2)
        def _():
            out_copy(g - 2).wait()

        out_copy(g).start()
        return 0

    stage_and_fire(0)
    lax.fori_loop(0, N_CHUNKS, step, 0)
    out_copy(N_CHUNKS - 2).wait()
    out_copy(N_CHUNKS - 1).wait()


@jax.jit
def _impl(idx, wt):
    mesh = plsc.VectorSubcoreMesh(core_axis_name="c", subcore_axis_name="s")

    tail16 = jax.lax.bitcast_convert_type(
        wt[:, TAIL_BASE:].T.astype(jnp.bfloat16).reshape(64, 16, 2),
        jnp.int32,
    ).reshape(8, 128)

    packed = pl.kernel(
        _cast_body,
        out_type=jax.ShapeDtypeStruct((PACKED_ROWS, 128), jnp.int32),
        mesh=mesh,
        scratch_types=[
            pltpu.VMEM((2, DIM, 128), jnp.float32),
            pltpu.VMEM((2, 16, 128), jnp.int32),
            pltpu.SemaphoreType.DMA((2,)),
            pltpu.SemaphoreType.DMA((2,)),
        ],
        compiler_params=pltpu.CompilerParams(
            needs_layout_passes=False, use_tc_tiling_on_sc=True
        ),
    )(wt, tail16)

    rows16 = packed.reshape(NUM_EMB, 16)

    out_i32 = pl.kernel(
        _gather_body,
        out_type=jax.ShapeDtypeStruct((N_ROWS, N_COLS, 16), jnp.int32),
        mesh=mesh,
        scratch_types=[
            pltpu.VMEM((2, R, N_COLS), jnp.int32),
            pltpu.VMEM((2, R, N_COLS, 16), jnp.int32),
            pltpu.SemaphoreType.DMA((2,)),
            pltpu.SemaphoreType.DMA((2,)),
        ],
        compiler_params=pltpu.CompilerParams(
            needs_layout_passes=False, use_tc_tiling_on_sc=False
        ),
    )(idx, rows16)

    out_pairs = jax.lax.bitcast_convert_type(out_i32, jnp.bfloat16)
    return out_pairs.reshape(N_ROWS, N_COLS, DIM)


def kernel(input, embedding_weight):
    return _impl(input.astype(jnp.int32), embedding_weight.T)


# speedup vs baseline: 2.4513x; 1.7301x over previous
import jax
import jax.numpy as jnp
from jax import lax
from jax.experimental import pallas as pl
from jax.experimental.pallas import tpu as pltpu
from jax.experimental.pallas import tpu_sc as plsc

DIM = 32
L = 16
NC = 2
NS = 16
NW = NC * NS

NUM_EMB = 1000000
N_TC = NUM_EMB // 128
TC_PER_W = N_TC // NW + 1
PACKED_ROWS = NUM_EMB // 8
TAIL_BASE = N_TC * 128

N_ROWS = 4096
N_COLS = 200
ROWS_PER_W = N_ROWS // NW
R = 8
N_CHUNKS = ROWS_PER_W // R
CR = R * N_COLS


def _cast_body(wt_hbm, tail_hbm, out_hbm, t_v, o_v, isem, osem):
    wid = lax.axis_index("s") * NC + lax.axis_index("c")

    lanes = lax.iota(jnp.int32, L)
    row_half = lanes // 8
    col_base = (lanes % 8) * L
    colvs = [col_base + k for k in range(L)]

    def tcg(t):
        return t * NW + wid

    def fire(t):
        p = t % 2

        @pl.when(tcg(t) < N_TC)
        def _():
            col0 = pl.multiple_of(tcg(t) * 128, 128)
            pltpu.async_copy(
                wt_hbm.at[:, pl.ds(col0, 128)], t_v.at[p], isem.at[p]
            )

    def wait_in(t):
        p = t % 2

        @pl.when(tcg(t) < N_TC)
        def _():
            pltpu.make_async_copy(
                wt_hbm.at[:, pl.ds(0, 128)], t_v.at[p], isem.at[p]
            ).wait()

    def convert(t):
        p = t % 2
        ov = o_v.at[p]

        def lgbody(lg, _):
            c0 = pl.multiple_of(lg * L, L)
            rowv = row_half + 2 * lg
            for k in range(L):
                a = t_v[p, 2 * k, pl.ds(c0, L)]
                b = t_v[p, 2 * k + 1, pl.ds(c0, L)]
                pk = plsc.pack(a, b, format=plsc.PackFormat.INTERLEAVED)
                plsc.store_scatter(
                    ov, [rowv, colvs[k]], plsc.bitcast(pk, jnp.int32)
                )
            return 0

        lax.fori_loop(0, 8, lgbody, 0)

    def out_copy(t):
        p = t % 2
        row0 = pl.multiple_of(tcg(t) * 16, 8)
        return pltpu.make_async_copy(
            o_v.at[p], out_hbm.at[pl.ds(row0, 16)], osem.at[p]
        )

    def step(t, _):
        fire(t + 1)
        wait_in(t)

        @pl.when(tcg(t) >= 2 * NW)
        def _():
            out_copy(t - 2).wait()

        @pl.when(tcg(t) < N_TC)
        def _():
            convert(t)
            out_copy(t).start()

        return 0

    fire(0)
    lax.fori_loop(0, TC_PER_W, step, 0)

    @pl.when(tcg(TC_PER_W - 2) < N_TC)
    def _():
        out_copy(TC_PER_W - 2).wait()

    @pl.when(tcg(TC_PER_W - 1) < N_TC)
    def _():
        out_copy(TC_PER_W - 1).wait()

    @pl.when(wid == 4)
    def _():
        pltpu.sync_copy(tail_hbm, o_v.at[0, pl.ds(0, 8)])
        pltpu.sync_copy(
            o_v.at[0, pl.ds(0, 8)], out_hbm.at[pl.ds(PACKED_ROWS - 8, 8)]
        )


def _gather_body(
    idx_hbm, table_hbm, out_hbm, idx_v, idxt_v, g_v, o_v, gsem, osem
):
    wid = lax.axis_index("s") * NC + lax.axis_index("c")
    row0 = pl.multiple_of(wid * 128, 128)

    lanes = lax.iota(jnp.int32, L)
    lanevs = [lanes + 16 * lg for lg in range(8)]

    pltpu.sync_copy(idx_hbm.at[pl.ds(row0, 128)], idx_v)

    def extract_idx(c):
        p = c % 2
        cv = jnp.full((L,), c, dtype=jnp.int32)
        for lg in range(8):
            idxt_v[p, pl.ds(lg * L, L)] = plsc.load_gather(
                idx_v, [lanevs[lg], cv]
            )

    def fire(c):
        p = c % 2
        pltpu.async_copy(table_hbm.at[idxt_v.at[p]], g_v.at[p], gsem.at[p])

    def drain(c):
        p = c % 2
        pltpu.make_async_copy(
            table_hbm.at[idxt_v.at[p]], g_v.at[p], gsem.at[p]
        ).wait()

    def transpose(c):
        p = c % 2
        gv = g_v.at[p]
        for k in range(16):
            kv = jnp.full((L,), k, dtype=jnp.int32)
            for lg in range(8):
                o_v[p, k // 4, k % 4, pl.ds(lg * L, L)] = plsc.load_gather(
                    gv, [lanevs[lg], kv]
                )

    def out_copy(c):
        p = c % 2
        return tuple(
            pltpu.make_async_copy(
                o_v.at[p, tr], out_hbm.at[c, tr, wid], osem.at[p]
            )
            for tr in range(4)
        )

    def step(c, _):
        @pl.when(c + 1 < N_COLS)
        def _():
            extract_idx(c + 1)
            fire(c + 1)

        drain(c)

        @pl.when(c >= 2)
        def _():
            for cp in out_copy(c - 2):
                cp.wait()

        transpose(c)
        for cp in out_copy(c):
            cp.start()
        return 0

    extract_idx(0)
    fire(0)
    lax.fori_loop(0, N_COLS, step, 0)
    for cp in out_copy(N_COLS - 2):
        cp.wait()
    for cp in out_copy(N_COLS - 1):
        cp.wait()


@jax.jit
def _impl(idx, wt):
    mesh = plsc.VectorSubcoreMesh(core_axis_name="c", subcore_axis_name="s")

    tail16 = jax.lax.bitcast_convert_type(
        wt[:, TAIL_BASE:].T.astype(jnp.bfloat16).reshape(64, 16, 2),
        jnp.int32,
    ).reshape(8, 128)

    packed = pl.kernel(
        _cast_body,
        out_type=jax.ShapeDtypeStruct((PACKED_ROWS, 128), jnp.int32),
        mesh=mesh,
        scratch_types=[
            pltpu.VMEM((2, DIM, 128), jnp.float32),
            pltpu.VMEM((2, 16, 128), jnp.int32),
            pltpu.SemaphoreType.DMA((2,)),
            pltpu.SemaphoreType.DMA((2,)),
        ],
        compiler_params=pltpu.CompilerParams(
            needs_layout_passes=False, use_tc_tiling_on_sc=True
        ),
    )(wt, tail16)

    rows16 = packed.reshape(NUM_EMB, 16)

    out5d = pl.kernel(
        _gather_body,
        out_type=jax.ShapeDtypeStruct((N_COLS, 4, NW, 4, 128), jnp.int32),
        mesh=mesh,
        scratch_types=[
            pltpu.VMEM((128, N_COLS), jnp.int32),
            pltpu.VMEM((2, 128), jnp.int32),
            pltpu.VMEM((2, 128, 16), jnp.int32),
            pltpu.VMEM((2, 4, 4, 128), jnp.int32),
            pltpu.SemaphoreType.DMA((2,)),
            pltpu.SemaphoreType.DMA((2,)),
        ],
        compiler_params=pltpu.CompilerParams(
            needs_layout_passes=False, use_tc_tiling_on_sc=False
        ),
    )(idx, rows16)

    out_pairs = jax.lax.bitcast_convert_type(out5d, jnp.bfloat16)
    out = out_pairs.transpose(2, 4, 0, 1, 3, 5)
    return out.reshape(N_ROWS, N_COLS, DIM)


def kernel(input, embedding_weight):
    return _impl(input.astype(jnp.int32), embedding_weight.T)
